# baseline (speedup 1.0000x reference)
import jax
import jax.numpy as jnp
from jax import lax
from jax.experimental import pallas as pl
from jax.experimental.pallas import tpu as pltpu
from jax.experimental.pallas import tpu_sc as plsc

EMB = 32
BATCH = 16384
ROWS_PER_128 = 128 // EMB
NC = 2
NS = 16
L = 16
NW = NC * NS
BPW = BATCH // NW
CH = 128
NCH = BPW // CH
GPC = CH // L


def _body(uid_hbm, aid_hbm, uw_hbm, aw_hbm, out_hbm,
          uidx_v, aidx_v, uhi_v, ahi_v,
          ub0, ab0, ub1, ab1, scores_v, sem0, sem1):
    wid = lax.axis_index("s") * NC + lax.axis_index("c")
    base = wid * BPW

    pltpu.sync_copy(uid_hbm.at[pl.ds(base, BPW)], uidx_v)
    pltpu.sync_copy(aid_hbm.at[pl.ds(base, BPW)], aidx_v)

    def stage(i, carry):
        s = pl.ds(i * L, L)
        uhi_v[s] = lax.shift_right_logical(uidx_v[s], 2)
        ahi_v[s] = lax.shift_right_logical(aidx_v[s], 2)
        return carry

    lax.fori_loop(0, BPW // L, stage, 0)

    bufs = [(ub0, ab0, sem0), (ub1, ab1, sem1)]

    def fire(c):
        ub, ab, sem = bufs[c % 2]
        s = pl.ds(c * CH, CH)
        cu = pltpu.async_copy(uw_hbm.at[uhi_v.at[s]], ub, sem)
        ca = pltpu.async_copy(aw_hbm.at[ahi_v.at[s]], ab, sem)
        return cu, ca

    lanes = lax.iota(jnp.int32, L)

    def compute(c):
        ub, ab, _ = bufs[c % 2]

        def group(g, carry):
            rows = lanes + g * L
            s = pl.ds(c * CH + g * L, L)
            ucol = (uidx_v[s] & (ROWS_PER_128 - 1)) * EMB
            acol = (aidx_v[s] & (ROWS_PER_128 - 1)) * EMB
            acc = jnp.zeros((L,), jnp.float32)
            for j in range(EMB):
                u = plsc.load_gather(ub, [rows, ucol + j])
                a = plsc.load_gather(ab, [rows, acol + j])
                acc = acc + u * a
            scores_v[s] = acc
            return carry

        lax.fori_loop(0, GPC, group, 0)

    pending = {0: fire(0)}
    for c in range(NCH):
        if c + 1 < NCH:
            pending[c + 1] = fire(c + 1)
        for cp in pending.pop(c):
            cp.wait()
        compute(c)

    pltpu.sync_copy(scores_v, out_hbm.at[pl.ds(base, BPW)])


@jax.jit
def kernel(user_ids, anime_ids, user_emb_w, anime_emb_w):
    nu, na = user_emb_w.shape[0], anime_emb_w.shape[0]
    uw128 = user_emb_w.reshape(nu // ROWS_PER_128, 128)
    aw128 = anime_emb_w.reshape(na // ROWS_PER_128, 128)
    mesh = plsc.VectorSubcoreMesh(core_axis_name="c", subcore_axis_name="s")
    run = pl.kernel(
        _body,
        out_type=jax.ShapeDtypeStruct((BATCH,), jnp.float32),
        mesh=mesh,
        compiler_params=pltpu.CompilerParams(needs_layout_passes=False),
        scratch_types=[
            pltpu.VMEM((BPW,), jnp.int32),
            pltpu.VMEM((BPW,), jnp.int32),
            pltpu.VMEM((BPW,), jnp.int32),
            pltpu.VMEM((BPW,), jnp.int32),
            pltpu.VMEM((CH, 128), jnp.float32),
            pltpu.VMEM((CH, 128), jnp.float32),
            pltpu.VMEM((CH, 128), jnp.float32),
            pltpu.VMEM((CH, 128), jnp.float32),
            pltpu.VMEM((BPW,), jnp.float32),
            pltpu.SemaphoreType.DMA,
            pltpu.SemaphoreType.DMA,
        ],
    )
    return run(user_ids, anime_ids, uw128, aw128)


# speedup vs baseline: 1.0011x; 1.0011x over previous
import jax
import jax.numpy as jnp
from jax import lax
from jax.experimental import pallas as pl
from jax.experimental.pallas import tpu as pltpu
from jax.experimental.pallas import tpu_sc as plsc

EMB = 32
BATCH = 16384
NC = 2
NS = 16
L = 16
NW = NC * NS
BPW = BATCH // NW
CH = 128
NCH = BPW // CH
GPC = CH // L


def _body(uid_hbm, aid_hbm, uw_hbm, aw_hbm, out_hbm,
          uidx_v, aidx_v, ub0, ab0, ub1, ab1, scores_v, sem0, sem1):
    wid = lax.axis_index("s") * NC + lax.axis_index("c")
    base = wid * BPW

    pltpu.sync_copy(uid_hbm.at[pl.ds(base, BPW)], uidx_v)
    pltpu.sync_copy(aid_hbm.at[pl.ds(base, BPW)], aidx_v)

    bufs = [(ub0, ab0, sem0), (ub1, ab1, sem1)]

    def fire(c):
        ub, ab, sem = bufs[c % 2]
        s = pl.ds(c * CH, CH)
        cu = pltpu.async_copy(uw_hbm.at[uidx_v.at[s]], ub, sem)
        ca = pltpu.async_copy(aw_hbm.at[aidx_v.at[s]], ab, sem)
        return cu, ca

    lanes = lax.iota(jnp.int32, L)

    def compute(c):
        ub, ab, _ = bufs[c % 2]

        def group(g, carry):
            rows = lanes + g * L
            acc = jnp.zeros((L,), jnp.float32)
            for j in range(EMB):
                col = jnp.full((L,), j, jnp.int32)
                u = plsc.load_gather(ub, [rows, col])
                a = plsc.load_gather(ab, [rows, col])
                acc = acc + u * a
            scores_v[pl.ds(c * CH + g * L, L)] = acc
            return carry

        lax.fori_loop(0, GPC, group, 0)

    pending = {0: fire(0)}
    for c in range(NCH):
        if c + 1 < NCH:
            pending[c + 1] = fire(c + 1)
        for cp in pending.pop(c):
            cp.wait()
        compute(c)

    pltpu.sync_copy(scores_v, out_hbm.at[pl.ds(base, BPW)])


@jax.jit
def kernel(user_ids, anime_ids, user_emb_w, anime_emb_w):
    mesh = plsc.VectorSubcoreMesh(core_axis_name="c", subcore_axis_name="s")
    run = pl.kernel(
        _body,
        out_type=jax.ShapeDtypeStruct((BATCH,), jnp.float32),
        mesh=mesh,
        compiler_params=pltpu.CompilerParams(
            needs_layout_passes=False, use_tc_tiling_on_sc=False),
        scratch_types=[
            pltpu.VMEM((BPW,), jnp.int32),
            pltpu.VMEM((BPW,), jnp.int32),
            pltpu.VMEM((CH, EMB), jnp.float32),
            pltpu.VMEM((CH, EMB), jnp.float32),
            pltpu.VMEM((CH, EMB), jnp.float32),
            pltpu.VMEM((CH, EMB), jnp.float32),
            pltpu.VMEM((BPW,), jnp.float32),
            pltpu.SemaphoreType.DMA,
            pltpu.SemaphoreType.DMA,
        ],
    )
    return run(user_ids, anime_ids, user_emb_w, anime_emb_w)
